# Initial kernel scaffold; baseline (speedup 1.0000x reference)
import functools

import jax
import jax.numpy as jnp
from jax.experimental import pallas as pl

K_NN = 10


def _body(xb_ref, xa_ref, h_ref, *, n_rows, blk):
    xb = xb_ref[...]
    xa = xa_ref[...]
    d = xb.shape[1]

    sqb = jnp.sum(xb * xb, axis=1, keepdims=True)
    sqa_row = jax.lax.dot_general(
        jnp.ones((1, d), jnp.float32), xa * xa,
        (((1,), (1,)), ((), ())), preferred_element_type=jnp.float32)
    mm = jax.lax.dot_general(
        xb, xa, (((1,), (1,)), ((), ())),
        preferred_element_type=jnp.float32)
    dist = jnp.maximum(sqb + sqa_row - 2.0 * mm, 0.0)

    avg = jnp.sum(dist, axis=1, keepdims=True) * (1.0 / n_rows)

    bits = jax.lax.bitcast_convert_type(dist, jnp.int32)
    cols = jax.lax.broadcasted_iota(jnp.int32, dist.shape, 1)
    packed = (bits & jnp.int32(-8192)) | cols

    intmax = jnp.int32(2**31 - 1)
    sels = []
    p = packed
    for _ in range(K_NN):
        m = jnp.min(p, axis=1, keepdims=True)
        sels.append(m)
        p = jnp.where(p == m, intmax, p)
    sel = jnp.concatenate(sels, axis=1)

    idx = sel & jnp.int32(8191)
    dsel = jax.lax.bitcast_convert_type(sel & jnp.int32(-8192), jnp.float32)
    w = jnp.exp(-(dsel * dsel) / (avg * avg + 1e-12))

    idx_t = jnp.transpose(idx)
    w_t = jnp.transpose(w)
    rows = jax.lax.broadcasted_iota(jnp.int32, (n_rows, 1), 0)
    acc = jnp.zeros((n_rows, blk), jnp.float32)
    for j in range(K_NN):
        acc = acc + jnp.where(rows == idx_t[j:j + 1, :], w_t[j:j + 1, :], 0.0)
    h_ref[...] = acc


def kernel(inputs):
    x = inputs
    n, d = x.shape
    blk = 256
    grid = n // blk
    body = functools.partial(_body, n_rows=n, blk=blk)
    return pl.pallas_call(
        body,
        grid=(grid,),
        in_specs=[
            pl.BlockSpec((blk, d), lambda i: (i, 0)),
            pl.BlockSpec((n, d), lambda i: (0, 0)),
        ],
        out_specs=pl.BlockSpec((n, blk), lambda i: (0, i)),
        out_shape=jax.ShapeDtypeStruct((n, n), jnp.float32),
    )(x, x)


# speedup vs baseline: 1.7761x; 1.7761x over previous
import functools

import jax
import jax.numpy as jnp
from jax.experimental import pallas as pl

K_NN = 10


def _body(xb_ref, xa_ref, h_ref, *, n_rows, blk):
    xb = xb_ref[...]
    xa = xa_ref[...]
    d = xb.shape[1]

    sqb = jnp.sum(xb * xb, axis=1, keepdims=True)
    sqa_row = jax.lax.dot_general(
        jnp.ones((1, d), jnp.float32), xa * xa,
        (((1,), (1,)), ((), ())), precision=jax.lax.Precision.HIGHEST,
        preferred_element_type=jnp.float32)
    mm = jax.lax.dot_general(
        xb, xa, (((1,), (1,)), ((), ())),
        preferred_element_type=jnp.float32)
    dist = jnp.maximum(sqb + sqa_row - 2.0 * mm, 0.0)

    avg = jnp.sum(dist, axis=1, keepdims=True) * (1.0 / n_rows)

    cols = jax.lax.broadcasted_iota(jnp.int32, dist.shape, 1)
    intmax = jnp.int32(2**31 - 1)
    inf = jnp.float32(jnp.inf)
    vals = []
    idxs = []
    p = dist
    for _ in range(K_NN):
        m = jnp.min(p, axis=1, keepdims=True)
        pk = jnp.where(p == m, cols, intmax)
        j = jnp.min(pk, axis=1, keepdims=True)
        p = jnp.where(pk == j, inf, p)
        vals.append(m)
        idxs.append(j)
    dsel = jnp.concatenate(vals, axis=1)
    idx = jnp.concatenate(idxs, axis=1)
    w = jnp.exp(-(dsel * dsel) / (avg * avg + 1e-12))

    idx_t = jnp.transpose(idx)
    w_t = jnp.transpose(w)
    rows = jax.lax.broadcasted_iota(jnp.int32, (n_rows, 1), 0)
    acc = jnp.zeros((n_rows, blk), jnp.float32)
    for j in range(K_NN):
        acc = acc + jnp.where(rows == idx_t[j:j + 1, :], w_t[j:j + 1, :], 0.0)
    h_ref[...] = acc


def kernel(inputs):
    x = inputs
    n, d = x.shape
    blk = 256
    grid = n // blk
    body = functools.partial(_body, n_rows=n, blk=blk)
    return pl.pallas_call(
        body,
        grid=(grid,),
        in_specs=[
            pl.BlockSpec((blk, d), lambda i: (i, 0)),
            pl.BlockSpec((n, d), lambda i: (0, 0)),
        ],
        out_specs=pl.BlockSpec((n, blk), lambda i: (0, i)),
        out_shape=jax.ShapeDtypeStruct((n, n), jnp.float32),
    )(x, x)
